# baseline (speedup 1.0000x reference)
import functools

import jax
import jax.numpy as jnp
from jax import lax
from jax.experimental import pallas as pl
from jax.experimental.pallas import tpu as pltpu
from jax.experimental.pallas import tpu_sc as plsc

TOTAL = 10_000_000
BLK = 10_000
NBLK = TOTAL // BLK
LANES = 16
VECS = BLK // LANES
UNROLL = 5
NC = 2
NS = 16
NW = NC * NS
NPAIR = (NBLK + 2 * NW - 1) // (2 * NW)

_mesh = plsc.VectorSubcoreMesh(
    core_axis_name="c", subcore_axis_name="s", num_cores=NC, num_subcores=NS
)


def _wid():
    return lax.axis_index("s") * NC + lax.axis_index("c")


def _in_copies(x_hbm, f_hbm, xb, fb, sem, blk):
    cx = pltpu.make_async_copy(x_hbm.at[pl.ds(blk * BLK, BLK)], xb, sem)
    cf = pltpu.make_async_copy(f_hbm.at[pl.ds(blk * BLK, BLK)], fb, sem)
    return cx, cf


def _masked_block_sum(xb, fb, acc):
    def body(i, a):
        base = i * (LANES * UNROLL)
        for u in range(UNROLL):
            xv = xb[pl.ds(base + u * LANES, LANES)]
            fv = fb[pl.ds(base + u * LANES, LANES)]
            a = a + jnp.where(xv != 0, fv, 0.0)
        return a

    return lax.fori_loop(0, VECS // UNROLL, body, acc)


@functools.partial(
    pl.kernel,
    out_type=jax.ShapeDtypeStruct((NW * LANES,), jnp.float32),
    mesh=_mesh,
    compiler_params=pltpu.CompilerParams(needs_layout_passes=False),
    scratch_types=[
        pltpu.VMEM((BLK,), jnp.int32),
        pltpu.VMEM((BLK,), jnp.int32),
        pltpu.VMEM((BLK,), jnp.float32),
        pltpu.VMEM((BLK,), jnp.float32),
        pltpu.VMEM((LANES,), jnp.float32),
        pltpu.SemaphoreType.DMA,
        pltpu.SemaphoreType.DMA,
    ],
)
def _psum_kernel(x_hbm, f_hbm, out_hbm, xb0, xb1, fb0, fb1, accv, sem0, sem1):
    w = _wid()
    xbs, fbs, sems = (xb0, xb1), (fb0, fb1), (sem0, sem1)

    cx, cf = _in_copies(x_hbm, f_hbm, xbs[0], fbs[0], sems[0], w)
    cx.start()
    cf.start()
    accv[...] = jnp.zeros((LANES,), jnp.float32)

    def step(k, _):
        b0 = w + (2 * k) * NW
        b1 = b0 + NW
        b2 = b0 + 2 * NW

        @pl.when(b1 < NBLK)
        def _():
            cx1, cf1 = _in_copies(x_hbm, f_hbm, xbs[1], fbs[1], sems[1], b1)
            cx1.start()
            cf1.start()

        cx0, cf0 = _in_copies(x_hbm, f_hbm, xbs[0], fbs[0], sems[0], b0)
        cx0.wait()
        cf0.wait()
        accv[...] = _masked_block_sum(xbs[0], fbs[0], accv[...])

        @pl.when(b2 < NBLK)
        def _():
            cx2, cf2 = _in_copies(x_hbm, f_hbm, xbs[0], fbs[0], sems[0], b2)
            cx2.start()
            cf2.start()

        @pl.when(b1 < NBLK)
        def _():
            cx1, cf1 = _in_copies(x_hbm, f_hbm, xbs[1], fbs[1], sems[1], b1)
            cx1.wait()
            cf1.wait()
            accv[...] = _masked_block_sum(xbs[1], fbs[1], accv[...])

        return 0

    lax.fori_loop(0, NPAIR, step, 0)
    pltpu.sync_copy(accv, out_hbm.at[pl.ds(w * LANES, LANES)])


@functools.partial(
    pl.kernel,
    out_type=jax.ShapeDtypeStruct((TOTAL,), jnp.float32),
    mesh=_mesh,
    compiler_params=pltpu.CompilerParams(needs_layout_passes=False),
    scratch_types=[
        pltpu.VMEM((BLK,), jnp.int32),
        pltpu.VMEM((BLK,), jnp.int32),
        pltpu.VMEM((BLK,), jnp.float32),
        pltpu.VMEM((BLK,), jnp.float32),
        pltpu.VMEM((BLK,), jnp.float32),
        pltpu.VMEM((BLK,), jnp.float32),
        pltpu.VMEM((NW * LANES,), jnp.float32),
        pltpu.SemaphoreType.DMA,
        pltpu.SemaphoreType.DMA,
        pltpu.SemaphoreType.DMA,
        pltpu.SemaphoreType.DMA,
    ],
)
def _scale_kernel(
    x_hbm, f_hbm, p_hbm, y_hbm,
    xb0, xb1, fb0, fb1, yb0, yb1, pv,
    si0, si1, so0, so1,
):
    w = _wid()
    xbs, fbs, ybs = (xb0, xb1), (fb0, fb1), (yb0, yb1)
    sis, sos = (si0, si1), (so0, so1)

    pltpu.sync_copy(p_hbm, pv)
    acc = lax.fori_loop(
        0, NW, lambda j, a: a + pv[pl.ds(j * LANES, LANES)],
        jnp.zeros((LANES,), jnp.float32),
    )
    lanes = jnp.arange(LANES, dtype=jnp.int32)
    for sh in (1, 2, 4, 8):
        pv[pl.ds(0, LANES)] = acc
        acc = acc + plsc.load_gather(pv, [lanes ^ sh])
    inv = 1.0 / acc

    def out_copy(yb, blk, sem):
        return pltpu.make_async_copy(yb, y_hbm.at[pl.ds(blk * BLK, BLK)], sem)

    def compute(slot, blk):
        def body(i, _):
            base = i * (LANES * UNROLL)
            for u in range(UNROLL):
                xv = xbs[slot][pl.ds(base + u * LANES, LANES)]
                fv = fbs[slot][pl.ds(base + u * LANES, LANES)]
                ybs[slot][pl.ds(base + u * LANES, LANES)] = jnp.where(
                    xv != 0, fv * inv, 0.0
                )
            return 0

        lax.fori_loop(0, VECS // UNROLL, body, 0)
        out_copy(ybs[slot], blk, sos[slot]).start()

    cx, cf = _in_copies(x_hbm, f_hbm, xbs[0], fbs[0], sis[0], w)
    cx.start()
    cf.start()

    def step(k, _):
        b0 = w + (2 * k) * NW
        b1 = b0 + NW
        b2 = b0 + 2 * NW

        @pl.when(b1 < NBLK)
        def _():
            cx1, cf1 = _in_copies(x_hbm, f_hbm, xbs[1], fbs[1], sis[1], b1)
            cx1.start()
            cf1.start()

        @pl.when(b0 >= 2 * NW)
        def _():
            out_copy(ybs[0], b0 - 2 * NW, sos[0]).wait()

        cx0, cf0 = _in_copies(x_hbm, f_hbm, xbs[0], fbs[0], sis[0], b0)
        cx0.wait()
        cf0.wait()
        compute(0, b0)

        @pl.when(b2 < NBLK)
        def _():
            cx2, cf2 = _in_copies(x_hbm, f_hbm, xbs[0], fbs[0], sis[0], b2)
            cx2.start()
            cf2.start()

        @pl.when(b1 < NBLK)
        def _():
            @pl.when(b1 >= 2 * NW)
            def _():
                out_copy(ybs[1], b1 - 2 * NW, sos[1]).wait()

            cx1, cf1 = _in_copies(x_hbm, f_hbm, xbs[1], fbs[1], sis[1], b1)
            cx1.wait()
            cf1.wait()
            compute(1, b1)

        return 0

    lax.fori_loop(0, NPAIR, step, 0)

    last0 = w + (NPAIR - 1) * 2 * NW
    out_copy(ybs[0], last0, sos[0]).wait()
    last1 = last0 + NW

    @pl.when(last1 < NBLK)
    def _():
        out_copy(ybs[1], last1, sos[1]).wait()


LN = 128
ROWS = TOTAL // LN
BR = 1024
GRID = (ROWS + BR - 1) // BR
REM = ROWS - (GRID - 1) * BR


def _tc_body(x_ref, f_ref, y_ref, cache, accv, inv_s):
    ph = pl.program_id(0)
    i = pl.program_id(1)

    @pl.when(ph == 0)
    def _():
        @pl.when(i == 0)
        def _():
            accv[...] = jnp.zeros((8, LN), jnp.float32)

        xb = x_ref[...]
        fb = f_ref[...]

        @pl.when(i != GRID - 1)
        def _():
            w = jnp.where(xb != 0, fb, 0.0)
            cache[pl.ds(i * BR, BR), :] = w
            accv[...] += jnp.sum(w.reshape(BR // 8, 8, LN), axis=0)

        @pl.when(i == GRID - 1)
        def _():
            rows = lax.broadcasted_iota(jnp.int32, (BR, LN), 0)
            w = jnp.where((xb != 0) & (rows < REM), fb, 0.0)
            cache[pl.ds(i * BR, BR), :] = w
            accv[...] += jnp.sum(w.reshape(BR // 8, 8, LN), axis=0)

    @pl.when(ph == 1)
    def _():
        @pl.when(i == 0)
        def _():
            inv_s[0] = 1.0 / jnp.sum(accv[...])

        y_ref[...] = cache[pl.ds(i * BR, BR), :] * inv_s[0]


_tc_call = pl.pallas_call(
    _tc_body,
    grid=(2, GRID),
    in_specs=[
        pl.BlockSpec((BR, LN), lambda p, i: (i * (1 - p), 0)),
        pl.BlockSpec((BR, LN), lambda p, i: (i * (1 - p), 0)),
    ],
    out_specs=pl.BlockSpec((BR, LN), lambda p, i: (i * p, 0)),
    out_shape=jax.ShapeDtypeStruct((ROWS, LN), jnp.float32),
    scratch_shapes=[
        pltpu.VMEM((GRID * BR, LN), jnp.float32),
        pltpu.VMEM((8, LN), jnp.float32),
        pltpu.SMEM((1,), jnp.float32),
    ],
)


def _tc2_body(x_ref, f_ref, y_ref, accv, inv_s):
    ph = pl.program_id(0)
    i = pl.program_id(1)

    @pl.when(ph == 0)
    def _():
        @pl.when(i == 0)
        def _():
            accv[...] = jnp.zeros((8, LN), jnp.float32)

        xb = x_ref[...]
        fb = f_ref[...]

        @pl.when(i != GRID - 1)
        def _():
            w = jnp.where(xb != 0, fb, 0.0)
            accv[...] += jnp.sum(w.reshape(BR // 8, 8, LN), axis=0)

        @pl.when(i == GRID - 1)
        def _():
            rows = lax.broadcasted_iota(jnp.int32, (BR, LN), 0)
            w = jnp.where((xb != 0) & (rows < REM), fb, 0.0)
            accv[...] += jnp.sum(w.reshape(BR // 8, 8, LN), axis=0)

    @pl.when(ph == 1)
    def _():
        @pl.when(i == 0)
        def _():
            inv_s[0] = 1.0 / jnp.sum(accv[...])

        y_ref[...] = jnp.where(x_ref[...] != 0, f_ref[...] * inv_s[0], 0.0)


_tc2_call = pl.pallas_call(
    _tc2_body,
    grid=(2, GRID),
    in_specs=[
        pl.BlockSpec((BR, LN), lambda p, i: (i, 0)),
        pl.BlockSpec((BR, LN), lambda p, i: (i, 0)),
    ],
    out_specs=pl.BlockSpec((BR, LN), lambda p, i: (i * p, 0)),
    out_shape=jax.ShapeDtypeStruct((ROWS, LN), jnp.float32),
    scratch_shapes=[
        pltpu.VMEM((8, LN), jnp.float32),
        pltpu.SMEM((1,), jnp.float32),
    ],
)


def kernel(t, x, f):
    del t
    y2d = _tc2_call(x.reshape(ROWS, LN), f.reshape(ROWS, LN))
    return y2d.reshape(TOTAL)


# speedup vs baseline: 3.3040x; 3.3040x over previous
import jax
import jax.numpy as jnp
from jax import lax
from jax.experimental import pallas as pl
from jax.experimental.pallas import tpu as pltpu

TOTAL = 10_000_000
LN = 128
ROWS = TOTAL // LN
BR = 1024
GRID = (ROWS + BR - 1) // BR
REM = ROWS - (GRID - 1) * BR
KR = 11
CH = GRID // KR


def _body(x_hbm, f_hbm, y_hbm, xr, fr, cache, accv, *sems):
    isems = sems[:KR]
    osems = sems[KR:]

    def in_descs(b, j, rows):
        cx = pltpu.make_async_copy(
            x_hbm.at[pl.ds(b * BR, rows), :], xr.at[pl.ds(j * BR, rows), :],
            isems[j])
        cf = pltpu.make_async_copy(
            f_hbm.at[pl.ds(b * BR, rows), :], fr.at[pl.ds(j * BR, rows), :],
            isems[j])
        return cx, cf

    def start_in(b, j, rows):
        cx, cf = in_descs(b, j, rows)
        cx.start()
        cf.start()

    def wait_in(b, j, rows):
        cx, cf = in_descs(b, j, rows)
        cx.wait()
        cf.wait()

    def out_desc(b, j, rows):
        return pltpu.make_async_copy(
            cache.at[pl.ds(b * BR, rows), :], y_hbm.at[pl.ds(b * BR, rows), :],
            osems[j])

    accv[...] = jnp.zeros((8, LN), jnp.float32)
    for j in range(KR):
        start_in(j, j, BR)

    def p0_step(c, _):
        for j in range(KR):
            b = c * KR + j
            if j == KR - 1:
                @pl.when(c == CH - 1)
                def _():
                    wait_in(b, j, REM)

                @pl.when(c != CH - 1)
                def _():
                    wait_in(b, j, BR)
            else:
                wait_in(b, j, BR)

            w = jnp.where(xr[pl.ds(j * BR, BR), :] != 0,
                          fr[pl.ds(j * BR, BR), :], 0.0)
            cache[pl.ds(b * BR, BR), :] = w
            accv[...] += jnp.sum(w.reshape(BR // 8, 8, LN), axis=0)

            if j == KR - 1:
                @pl.when(c == CH - 2)
                def _():
                    xr[pl.ds(j * BR + REM, BR - REM), :] = jnp.zeros(
                        (BR - REM, LN), jnp.int32)
                    start_in(b + KR, j, REM)

                @pl.when(c < CH - 2)
                def _():
                    start_in(b + KR, j, BR)
            else:
                @pl.when(c < CH - 1)
                def _():
                    start_in(b + KR, j, BR)
        return 0

    lax.fori_loop(0, CH, p0_step, 0)
    inv = 1.0 / jnp.sum(accv[...])

    def p1_step(c, _):
        for j in range(0, KR - 1, 2):
            b = c * KR + j

            @pl.when(c > 0)
            def _():
                out_desc(b - KR, j, 2 * BR).wait()

            cache[pl.ds(b * BR, 2 * BR), :] = (
                cache[pl.ds(b * BR, 2 * BR), :] * inv)
            out_desc(b, j, 2 * BR).start()

        j = KR - 1
        b = c * KR + j

        @pl.when(c > 0)
        def _():
            out_desc(b - KR, j, BR).wait()

        cache[pl.ds(b * BR, BR), :] = cache[pl.ds(b * BR, BR), :] * inv

        @pl.when(c == CH - 1)
        def _():
            out_desc(b, j, REM).start()

        @pl.when(c != CH - 1)
        def _():
            out_desc(b, j, BR).start()
        return 0

    lax.fori_loop(0, CH, p1_step, 0)
    for j in range(0, KR - 1, 2):
        out_desc((CH - 1) * KR + j, j, 2 * BR).wait()
    out_desc((CH - 1) * KR + KR - 1, KR - 1, REM).wait()


_call = pl.pallas_call(
    _body,
    in_specs=[
        pl.BlockSpec(memory_space=pltpu.MemorySpace.HBM),
        pl.BlockSpec(memory_space=pltpu.MemorySpace.HBM),
    ],
    out_specs=pl.BlockSpec(memory_space=pltpu.MemorySpace.HBM),
    out_shape=jax.ShapeDtypeStruct((ROWS, LN), jnp.float32),
    scratch_shapes=[
        pltpu.VMEM((KR * BR, LN), jnp.int32),
        pltpu.VMEM((KR * BR, LN), jnp.float32),
        pltpu.VMEM((GRID * BR, LN), jnp.float32),
        pltpu.VMEM((8, LN), jnp.float32),
    ] + [pltpu.SemaphoreType.DMA] * (2 * KR),
)


def kernel(t, x, f):
    del t
    y2d = _call(x.reshape(ROWS, LN), f.reshape(ROWS, LN))
    return y2d.reshape(TOTAL)
